# baseline (speedup 1.0000x reference)
import functools

import numpy as np
import jax
from jax import lax
import jax.numpy as jnp
from jax.experimental import pallas as pl
from jax.experimental.pallas import tpu as pltpu
from jax.experimental.pallas import tpu_sc as plsc

_B = 16
_N = 8192
_NPTS = 1024
_K = 32
_L = 16
_R = _N // _L


_ROT = [[13, 15, 26, 6], [17, 29, 16, 24]]


def _rotl(x, r):
    return ((x << np.uint32(r)) | (x >> np.uint32(32 - r))).astype(np.uint32)


def _tf2x32(k0, k1, x0, x1):
    k0 = np.asarray(k0, np.uint32)
    k1 = np.asarray(k1, np.uint32)
    ks = [k0, k1, (k0 ^ k1 ^ np.uint32(0x1BD11BDA)).astype(np.uint32)]
    x = [(np.asarray(x0, np.uint32) + ks[0]).astype(np.uint32),
         (np.asarray(x1, np.uint32) + ks[1]).astype(np.uint32)]
    for i in range(5):
        for r in _ROT[i % 2]:
            x[0] = (x[0] + x[1]).astype(np.uint32)
            x[1] = _rotl(x[1], r)
            x[1] = (x[1] ^ x[0]).astype(np.uint32)
        x[0] = (x[0] + ks[(i + 1) % 3]).astype(np.uint32)
        x[1] = (x[1] + ks[(i + 2) % 3] + np.uint32(i + 1)).astype(np.uint32)
    return x[0], x[1]


def _rng_tables():
    seed = np.array([0, 42], np.uint32)
    bs = np.arange(_B, dtype=np.uint32)
    k0, k1 = _tf2x32(seed[0], seed[1], np.zeros(_B, np.uint32), bs)
    hb = np.zeros((_B, _NPTS), np.uint32)
    lb = np.zeros((_B, _NPTS), np.uint32)
    z = np.zeros(_B, np.uint32)
    for i in range(_NPTS):
        a0, a1 = _tf2x32(k0, k1, z, z)
        b0, b1 = _tf2x32(k0, k1, z, z + np.uint32(1))
        c0, c1 = _tf2x32(b0, b1, z, z)
        d0, d1 = _tf2x32(b0, b1, z, z + np.uint32(1))
        e0, e1 = _tf2x32(c0, c1, z, z)
        f0, f1 = _tf2x32(d0, d1, z, z)
        hb[:, i] = e0 ^ e1
        lb[:, i] = f0 ^ f1
        k0, k1 = a0, a1
    return hb, lb


_HB_NP, _LB_NP = _rng_tables()

_INF = float(np.inf)
_BIG = 1 << 30


def _splat_i(s):
    return jnp.zeros((_L,), jnp.int32) + s


def _splat_f(s):
    return jnp.zeros((_L,), jnp.float32) + s


def _bar(v):
    return plsc.bitcast(plsc.bitcast(v, jnp.int32), jnp.float32)


def _sc_body(x_hbm, y_hbm, z_hbm, hb_hbm, lb_hbm, idx_hbm, pts_hbm,
             xv, yv, zv, uv, dv, hbv, lbv, idsv, ptsv, sem):
    core = lax.axis_index("c")
    sub = lax.axis_index("s")
    b = sub * 2 + core

    @pl.when(sub < 8)
    def _():
        lane = lax.broadcasted_iota(jnp.int32, (_L,), 0)

        pltpu.sync_copy(x_hbm.at[b], xv)
        pltpu.sync_copy(y_hbm.at[b], yv)
        pltpu.sync_copy(z_hbm.at[b], zv)
        pltpu.sync_copy(hb_hbm.at[b], hbv)
        pltpu.sync_copy(lb_hbm.at[b], lbv)

        def _init(t, carry):
            uv[pl.ds(t * _L, _L)] = jnp.zeros((_L,), jnp.int32)
            return carry

        lax.fori_loop(0, _R, _init, 0)

        def step(i, cid):
            def sc1(t, carry):
                cnt_v, umin_v = carry
                v = uv[pl.ds(t * _L, _L)]
                mk = (v == cid)
                cnt_v = cnt_v + plsc.all_reduce_population_count(mk)
                umin_v = jnp.minimum(umin_v, v)
                return cnt_v, umin_v

            cnt_v, umin_v = lax.fori_loop(
                0, _R, sc1, (_splat_i(0), _splat_i(_BIG)))
            cnt0 = jnp.min(cnt_v)
            umin = jnp.min(umin_v)

            def _recount(_):
                def sc2(t, acc):
                    v = uv[pl.ds(t * _L, _L)]
                    return acc + plsc.all_reduce_population_count(v == umin)

                return umin, jnp.min(lax.fori_loop(0, _R, sc2, _splat_i(0)))

            def _keep(_):
                return cid, cnt0

            cid, cnt = lax.cond(cnt0 == 0, _recount, _keep, 0)

            span = jnp.maximum(cnt, 1).astype(jnp.uint32)
            hbu = plsc.bitcast(plsc.load_gather(hbv, [_splat_i(i)]),
                               jnp.uint32)
            lbu = plsc.bitcast(plsc.load_gather(lbv, [_splat_i(i)]),
                               jnp.uint32)
            m1 = jnp.uint32(1 << 16) % span
            mult = (m1 * m1) % span
            r_v = ((hbu % span) * mult + (lbu % span)) % span
            r = jnp.min(r_v.astype(jnp.int32))
            want = r + 1

            def sc3(t, carry):
                acc, found = carry
                v = uv[pl.ds(t * _L, _L)]
                mk = (v == cid)
                pc = jnp.min(plsc.all_reduce_population_count(mk))
                cs = plsc.cumsum(mk.astype(jnp.int32))
                hitl = (cs == (want - acc)) & mk
                l = jnp.min(plsc.all_reduce_ffs(hitl))
                crossing = (acc < want) & (acc + pc >= want)
                found = jnp.where(crossing, t * _L + l, found)
                return acc + pc, found

            _, index = lax.fori_loop(0, _R, sc3, (0, _BIG))

            iv = _splat_i(index)
            px = plsc.load_gather(xv, [iv])
            py = plsc.load_gather(yv, [iv])
            pz = plsc.load_gather(zv, [iv])

            def sc4(t, carry):
                bd, bi = carry
                dx = xv[pl.ds(t * _L, _L)] - px
                dy = yv[pl.ds(t * _L, _L)] - py
                dz = zv[pl.ds(t * _L, _L)] - pz
                sx = _bar(dx * dx)
                sy = _bar(dy * dy)
                sz = _bar(dz * dz)
                d = _bar(sx + sy) + sz
                dv[pl.ds(t * _L, _L)] = d
                jv = lane + t * _L
                take = d < bd
                bd = jnp.where(take, d, bd)
                bi = jnp.where(take, jv, bi)
                return bd, bi

            bd, bi = lax.fori_loop(0, _R, sc4, (_splat_f(_INF),
                                                _splat_i(_BIG)))

            def pop(k, carry):
                bd, bi = carry
                m = jnp.min(bd)
                g = jnp.min(jnp.where(bd == m, bi, _BIG))
                ms = _splat_f(m)
                gs = _splat_i(g)
                plsc.store_scatter(idsv, [_splat_i(i * _K + k)], gs,
                                   mask=(lane == 0))
                plsc.addupdate_scatter(uv, [gs], _splat_i(1), mask=(lane == 0))
                lstar = g & (_L - 1)

                def rs(t, c):
                    cd, ci = c
                    addr = t * (_L * _L) + lane * _L + lstar
                    gd = plsc.load_gather(dv, [addr])
                    ok = (gd > ms) | ((gd == ms) & (addr > gs))
                    lt = ok & ((gd < cd) | ((gd == cd) & (addr < ci)))
                    cd = jnp.where(lt, gd, cd)
                    ci = jnp.where(lt, addr, ci)
                    return cd, ci

                cd, ci = lax.fori_loop(0, _R // _L, rs,
                                       (_splat_f(_INF), _splat_i(_BIG)))
                nm = jnp.min(cd)
                ng = jnp.min(jnp.where(cd == nm, ci, _BIG))
                at = (lane == lstar)
                bd = jnp.where(at, _splat_f(nm), bd)
                bi = jnp.where(at, _splat_i(ng), bi)
                return bd, bi

            lax.fori_loop(0, _K, pop, (bd, bi))

            plsc.addupdate_scatter(uv, [iv], _splat_i(100), mask=(lane == 0))

            pv = jnp.where(lane == 0, px,
                           jnp.where(lane == 1, py, pz))
            plsc.store_scatter(ptsv, [_splat_i(3 * i) + lane], pv, mask=(lane < 3))
            return cid

        lax.fori_loop(0, _NPTS, step, 0)

        pltpu.sync_copy(idsv, idx_hbm.at[b])
        pltpu.sync_copy(ptsv, pts_hbm.at[b])


@jax.jit
def _run_sc(xyz1):
    x = xyz1[:, :, 0]
    y = xyz1[:, :, 1]
    z = xyz1[:, :, 2]
    hb = jnp.asarray(_HB_NP.astype(np.int32))
    lb = jnp.asarray(_LB_NP.astype(np.int32))
    mesh = plsc.VectorSubcoreMesh(core_axis_name="c", subcore_axis_name="s")
    kfn = functools.partial(
        pl.kernel,
        mesh=mesh,
        compiler_params=pltpu.CompilerParams(needs_layout_passes=False),
        out_type=[
            jax.ShapeDtypeStruct((_B, _NPTS * _K), jnp.int32),
            jax.ShapeDtypeStruct((_B, _NPTS * 3), jnp.float32),
        ],
        scratch_types=[
            pltpu.VMEM((_N,), jnp.float32),
            pltpu.VMEM((_N,), jnp.float32),
            pltpu.VMEM((_N,), jnp.float32),
            pltpu.VMEM((_N,), jnp.int32),
            pltpu.VMEM((_N,), jnp.float32),
            pltpu.VMEM((_NPTS,), jnp.int32),
            pltpu.VMEM((_NPTS,), jnp.int32),
            pltpu.VMEM((_NPTS * _K,), jnp.int32),
            pltpu.VMEM((_NPTS * 3,), jnp.float32),
            pltpu.SemaphoreType.DMA,
        ],
    )
    idx, pts = kfn(_sc_body)(x, y, z, hb, lb)
    idx = jnp.reshape(idx, (_B, _NPTS, _K))[..., None]
    pts = jnp.reshape(pts, (_B, _NPTS, 3))
    return idx, pts


def kernel(xyz1):
    return _run_sc(jax.lax.stop_gradient(xyz1))


# speedup vs baseline: 1.5715x; 1.5715x over previous
import functools

import numpy as np
import jax
import jax.numpy as jnp
from jax.experimental import pallas as pl
from jax.experimental.pallas import tpu as pltpu

_B = 16
_N = 8192
_NPTS = 1024
_K = 32


_ROT = [[13, 15, 26, 6], [17, 29, 16, 24]]


def _rotl(x, r):
    return ((x << np.uint32(r)) | (x >> np.uint32(32 - r))).astype(np.uint32)


def _tf2x32(k0, k1, x0, x1):
    k0 = np.asarray(k0, np.uint32)
    k1 = np.asarray(k1, np.uint32)
    ks = [k0, k1, (k0 ^ k1 ^ np.uint32(0x1BD11BDA)).astype(np.uint32)]
    x = [(np.asarray(x0, np.uint32) + ks[0]).astype(np.uint32),
         (np.asarray(x1, np.uint32) + ks[1]).astype(np.uint32)]
    for i in range(5):
        for r in _ROT[i % 2]:
            x[0] = (x[0] + x[1]).astype(np.uint32)
            x[1] = _rotl(x[1], r)
            x[1] = (x[1] ^ x[0]).astype(np.uint32)
        x[0] = (x[0] + ks[(i + 1) % 3]).astype(np.uint32)
        x[1] = (x[1] + ks[(i + 2) % 3] + np.uint32(i + 1)).astype(np.uint32)
    return x[0], x[1]


def _rng_tables():
    seed = np.array([0, 42], np.uint32)
    bs = np.arange(_B, dtype=np.uint32)
    k0, k1 = _tf2x32(seed[0], seed[1], np.zeros(_B, np.uint32), bs)
    hb = np.zeros((_NPTS, _B), np.uint32)
    lb = np.zeros((_NPTS, _B), np.uint32)
    z = np.zeros(_B, np.uint32)
    for i in range(_NPTS):
        a0, a1 = _tf2x32(k0, k1, z, z)
        b0, b1 = _tf2x32(k0, k1, z, z + np.uint32(1))
        c0, c1 = _tf2x32(b0, b1, z, z)
        d0, d1 = _tf2x32(b0, b1, z, z + np.uint32(1))
        e0, e1 = _tf2x32(c0, c1, z, z)
        f0, f1 = _tf2x32(d0, d1, z, z)
        hb[i] = e0 ^ e1
        lb[i] = f0 ^ f1
        k0, k1 = a0, a1
    return hb, lb


_HB_NP, _LB_NP = _rng_tables()


def _mod(a, s):
    for k in range(17, -1, -1):
        t = s << k
        a = jnp.where(a >= t, a - t, a)
    return a


def _lane_cumsum(m):
    sh = 1
    while sh < _N:
        z = jnp.zeros((_B, sh), m.dtype)
        m = m + jnp.concatenate([z, m[:, :-sh]], axis=1)
        sh *= 2
    return m


def _body(npts, x_ref, y_ref, z_ref, hb_ref, lb_ref, idx_ref, pts_ref,
          used_ref):
    x = x_ref[...]
    y = y_ref[...]
    z = z_ref[...]
    iota = jax.lax.broadcasted_iota(jnp.int32, (_B, _N), 1)
    used_ref[...] = jnp.zeros((_B, _N), jnp.int32)
    inf = jnp.float32(np.inf)

    def step(i, cid):
        used = used_ref[...]
        mask0 = (used == cid)
        cnt0 = jnp.sum(mask0.astype(jnp.int32), axis=1, keepdims=True)
        umin = jnp.min(used, axis=1, keepdims=True)
        cid = jnp.where(cnt0 == 0, umin, cid)
        mask = (used == cid)
        cnt = jnp.sum(mask.astype(jnp.int32), axis=1, keepdims=True)
        span = jnp.maximum(cnt, 1)

        hbw = jnp.reshape(hb_ref[pl.ds(i, 1)], (_B, 1))
        lbw = jnp.reshape(lb_ref[pl.ds(i, 1)], (_B, 1))
        m1 = _mod(jnp.full((_B, 1), 1 << 16, jnp.int32), span)
        mult = _mod(m1 * m1, span)

        def _mod32(w):
            hi = jax.lax.shift_right_logical(w, 16)
            lo = jax.lax.bitwise_and(w, (1 << 16) - 1)
            him = _mod(hi, span)
            return _mod(him * m1 + lo, span)

        r = _mod(_mod32(hbw) * mult + _mod32(lbw), span)

        csum = _lane_cumsum(mask.astype(jnp.int32))
        hit = (csum == (r + 1)) & mask
        index = jnp.min(jnp.where(hit, iota, _N), axis=1, keepdims=True)

        sel = (iota == index)
        px = jnp.sum(jnp.where(sel, x, 0.0), axis=1, keepdims=True)
        py = jnp.sum(jnp.where(sel, y, 0.0), axis=1, keepdims=True)
        pz = jnp.sum(jnp.where(sel, z, 0.0), axis=1, keepdims=True)

        dx = x - px
        dy = y - py
        dz = z - pz
        d = dx * dx + dy * dy + dz * dz

        ids = []
        dw = d
        m = None
        j = None
        for _ in range(_K):
            m = jnp.min(dw, axis=1, keepdims=True)
            j = jnp.min(jnp.where(dw == m, iota, _N), axis=1, keepdims=True)
            ids.append(j)
            dw = jnp.where(iota == j, inf, dw)

        member = (d < m) | ((d == m) & (iota <= j))
        used = used + member.astype(jnp.int32) + jnp.where(sel, 100, 0)
        used_ref[...] = used

        idx_ref[pl.ds(i, 1)] = jnp.reshape(
            jnp.concatenate(ids, axis=1), (1, _B, _K))
        pts_ref[pl.ds(i, 1)] = jnp.reshape(
            jnp.concatenate([px, py, pz], axis=1), (1, _B, 3))
        return cid

    jax.lax.fori_loop(0, npts, step, jnp.zeros((_B, 1), jnp.int32))


@functools.partial(jax.jit, static_argnums=(1, 2))
def _run(xyz1, npts, interpret):
    x = xyz1[:, :, 0]
    y = xyz1[:, :, 1]
    z = xyz1[:, :, 2]
    hb = jnp.asarray(_HB_NP[:npts].astype(np.int32)).reshape(npts, _B, 1)
    lb = jnp.asarray(_LB_NP[:npts].astype(np.int32)).reshape(npts, _B, 1)
    idx, pts = pl.pallas_call(
        functools.partial(_body, npts),
        out_shape=[
            jax.ShapeDtypeStruct((npts, _B, _K), jnp.int32),
            jax.ShapeDtypeStruct((npts, _B, 3), jnp.float32),
        ],
        scratch_shapes=[pltpu.VMEM((_B, _N), jnp.int32)],
        interpret=interpret,
    )(x, y, z, hb, lb)
    idx = jnp.transpose(idx, (1, 0, 2))[..., None]
    pts = jnp.transpose(pts, (1, 0, 2))
    return idx, pts


def kernel(xyz1):
    return _run(jax.lax.stop_gradient(xyz1), _NPTS, False)
